# baseline (speedup 1.0000x reference)
import jax
import jax.numpy as jnp
from jax.experimental import pallas as pl

_B, _H, _W = 32, 56, 56
_NP, _E, _O = 4, 64, 256
_HW = _H * _W
_NB = 4


def _period_kernel(labels_ref, emb_ref, w_ref, b_ref, out_ref):
    i = pl.program_id(0)
    labels = labels_ref[pl.ds(i * _NB, _NB), 0]
    onehot = (labels[:, None] == jax.lax.broadcasted_iota(
        jnp.int32, (_NB, _NP), 1)).astype(jnp.float32)
    emb = jnp.dot(onehot, emb_ref[:], preferred_element_type=jnp.float32)
    rows = jax.lax.dot_general(
        emb, w_ref[:], (((1,), (1,)), ((), ())),
        preferred_element_type=jnp.float32) + b_ref[:]
    out_ref[:] = jnp.broadcast_to(rows[:, :, None], (_NB, _O, _HW))


def kernel(period_labels, spatial_size, emb_table, fc_w, fc_b):
    labels2d = period_labels.reshape(_B, 1).astype(jnp.int32)
    fcb2d = fc_b.reshape(1, _O)
    out = pl.pallas_call(
        _period_kernel,
        grid=(_B // _NB,),
        in_specs=[
            pl.BlockSpec((_B, 1), lambda i: (0, 0)),
            pl.BlockSpec((_NP, _E), lambda i: (0, 0)),
            pl.BlockSpec((_O, _E), lambda i: (0, 0)),
            pl.BlockSpec((1, _O), lambda i: (0, 0)),
        ],
        out_specs=pl.BlockSpec((_NB, _O, _HW), lambda i: (i, 0, 0)),
        out_shape=jax.ShapeDtypeStruct((_B, _O, _HW), jnp.float32),
    )(labels2d, emb_table, fc_w, fcb2d)
    return out.reshape(_B, _O, _H, _W)
